# baseline (speedup 1.0000x reference)
import jax
import jax.numpy as jnp
from jax.experimental import pallas as pl
from jax.experimental.pallas import tpu as pltpu

R = 128
V = 100000
CHUNK = 3200
NC = 32
NS = NC // 2
TILE_L = 640
NT_L = CHUNK // TILE_L

KEY0 = 0
KEY1 = 42

NEG_INF = float("-inf")
BIG_I32 = 2**31 - 1
INV_LN2 = 1.4426950408889634


def _threefry_bits(xlo):
    u32 = jnp.uint32
    ks0 = u32(KEY0)
    ks1 = u32(KEY1)
    ks2 = u32(KEY0 ^ KEY1 ^ 0x1BD11BDA)
    ks = (ks0, ks1, ks2)
    rot1 = (13, 15, 26, 6)
    rot2 = (17, 29, 16, 24)

    x0 = jnp.zeros_like(xlo) + ks0
    x1 = xlo + ks1
    for i in range(5):
        for r in (rot1 if i % 2 == 0 else rot2):
            x0 = x0 + x1
            x1 = (x1 << u32(r)) | (x1 >> u32(32 - r))
            x1 = x1 ^ x0
        x0 = x0 + ks[(i + 1) % 3]
        x1 = x1 + ks[(i + 2) % 3] + u32(i + 1)
    return x0 ^ x1


def _sample_kernel(logits_ref, t_ref, val_ref, idx_ref, acc_val, acc_chunk):
    h = pl.program_id(0)
    j = pl.program_id(1)
    jg = h * NS + j

    @pl.when(j == 0)
    def _init():
        acc_val[...] = jnp.full((R, CHUNK), NEG_INF, jnp.float32)
        acc_chunk[...] = jnp.zeros((R, CHUNK), jnp.int32)

    row_iota = jax.lax.broadcasted_iota(jnp.uint32, (8, TILE_L), 0)
    lane_iota = jax.lax.broadcasted_iota(jnp.uint32, (8, TILE_L), 1)
    lane_i32 = lane_iota.astype(jnp.int32)
    base = row_iota * jnp.uint32(V) + lane_iota
    chunk_col0 = jg * CHUNK

    for rg in range(R // 8):
        r0 = rg * 8
        s_tile = jnp.float32(INV_LN2) / t_ref[pl.ds(r0, 8), :]

        def lane_body(tl, _, r0=r0, s_tile=s_tile):
            c0 = tl * TILE_L
            xs = logits_ref[pl.ds(r0, 8), pl.ds(c0, TILE_L)]
            col0 = chunk_col0 + c0
            flat = base + jnp.uint32(r0 * V) + col0.astype(jnp.uint32)
            bits = _threefry_bits(flat)
            u = jax.lax.bitcast_convert_type(
                (bits >> jnp.uint32(9)) | jnp.uint32(0x3F800000),
                jnp.float32) - 1.0
            u = jnp.maximum(u, 0.0)
            n2 = jnp.maximum(-jnp.log2(1.0 - u), 1e-10)
            y = xs * s_tile - jnp.log2(n2)
            y = jnp.where(col0 + lane_i32 < V, y, NEG_INF)

            av = acc_val[pl.ds(r0, 8), pl.ds(c0, TILE_L)]
            take = y > av
            ac = acc_chunk[pl.ds(r0, 8), pl.ds(c0, TILE_L)]
            acc_chunk[pl.ds(r0, 8), pl.ds(c0, TILE_L)] = jnp.where(take, jg, ac)
            acc_val[pl.ds(r0, 8), pl.ds(c0, TILE_L)] = jnp.where(take, y, av)
            return 0

        jax.lax.fori_loop(0, NT_L, lane_body, 0)

    @pl.when(j == NS - 1)
    def _finalize():
        vals = acc_val[...]
        row_max = jnp.max(vals, axis=1, keepdims=True)
        full_lane = jax.lax.broadcasted_iota(jnp.int32, (R, CHUNK), 1)
        cols = acc_chunk[...] * CHUNK + full_lane
        cand = jnp.where(vals == row_max, cols, BIG_I32)
        val_ref[0] = row_max
        idx_ref[0] = jnp.min(cand, axis=1, keepdims=True)


def kernel(logits, temperatures):
    t2 = temperatures.reshape(R, 1)
    vals, idxs = pl.pallas_call(
        _sample_kernel,
        grid=(2, NS),
        in_specs=[
            pl.BlockSpec((R, CHUNK), lambda h, j: (0, h * NS + j)),
            pl.BlockSpec((R, 1), lambda h, j: (0, 0)),
        ],
        out_specs=[
            pl.BlockSpec((1, R, 1), lambda h, j: (h, 0, 0)),
            pl.BlockSpec((1, R, 1), lambda h, j: (h, 0, 0)),
        ],
        out_shape=[
            jax.ShapeDtypeStruct((2, R, 1), jnp.float32),
            jax.ShapeDtypeStruct((2, R, 1), jnp.int32),
        ],
        scratch_shapes=[
            pltpu.VMEM((R, CHUNK), jnp.float32),
            pltpu.VMEM((R, CHUNK), jnp.int32),
        ],
        compiler_params=pltpu.CompilerParams(
            dimension_semantics=("parallel", "arbitrary"),
        ),
    )(logits, t2)
    return jnp.where(vals[1, :, 0] > vals[0, :, 0], idxs[1, :, 0], idxs[0, :, 0])


# speedup vs baseline: 1.2570x; 1.2570x over previous
import jax
import jax.numpy as jnp
from jax.experimental import pallas as pl
from jax.experimental.pallas import tpu as pltpu

R = 128
V = 100000
CHUNK = 4096
NC = 25
TILE_R = 16
TILE_L = 512
NT = (R // TILE_R) * (CHUNK // TILE_L)

KEY1 = 42

NEG_INF = float("-inf")
BIG_I32 = 2**31 - 1
INV_LN2 = 1.4426950408889634


def _threefry_bits_key042(x1i):
    u32 = jnp.uint32
    ks0 = u32(0)
    ks1 = u32(KEY1)
    ks2 = u32(KEY1 ^ 0x1BD11BDA)
    ks = (ks0, ks1, ks2)
    rot1 = (13, 15, 26, 6)
    rot2 = (17, 29, 16, 24)

    x0 = x1i
    x1 = ((x1i << u32(13)) | (x1i >> u32(19))) ^ x0
    for r in rot1[1:]:
        x0 = x0 + x1
        x1 = (x1 << u32(r)) | (x1 >> u32(32 - r))
        x1 = x1 ^ x0
    x0 = x0 + ks[1]
    x1 = x1 + ks[2] + u32(1)
    for i in range(1, 5):
        for r in (rot1 if i % 2 == 0 else rot2):
            x0 = x0 + x1
            x1 = (x1 << u32(r)) | (x1 >> u32(32 - r))
            x1 = x1 ^ x0
        x0 = x0 + ks[(i + 1) % 3]
        x1 = x1 + ks[(i + 2) % 3] + u32(i + 1)
    return x0 ^ x1


def _sample_kernel(logits_ref, t_ref, val_ref, idx_ref,
                   acc_val, acc_chunk, s_ref):
    j = pl.program_id(0)

    @pl.when(j == 0)
    def _init():
        acc_val[...] = jnp.full((R, CHUNK), NEG_INF, jnp.float32)
        acc_chunk[...] = jnp.zeros((R, CHUNK), jnp.int32)
        s_ref[...] = jnp.float32(INV_LN2) / t_ref[...]

    row_iota = jax.lax.broadcasted_iota(jnp.uint32, (TILE_R, TILE_L), 0)
    lane_iota = jax.lax.broadcasted_iota(jnp.uint32, (TILE_R, TILE_L), 1)
    base = row_iota * jnp.uint32(V) + lane_iota
    lane_i32 = lane_iota.astype(jnp.int32)
    chunk_col0 = j * CHUNK

    def tile_at(t):
        r0 = (t >> 3) * TILE_R
        c0 = (t & 7) * TILE_L
        return r0, c0

    def compute_y(r0, c0):
        xs = logits_ref[pl.ds(r0, TILE_R), pl.ds(c0, TILE_L)]
        s_tile = s_ref[pl.ds(r0, TILE_R), :]
        col0 = chunk_col0 + c0
        x1i = base + (r0 * V + col0 + KEY1).astype(jnp.uint32)
        bits = _threefry_bits_key042(x1i)
        u = jax.lax.bitcast_convert_type(
            (bits >> jnp.uint32(9)) | jnp.uint32(0x3F800000),
            jnp.float32) - 1.0
        n2 = jnp.maximum(-jnp.log2(1.0 - u), 1e-10)
        return xs * s_tile - jnp.log2(n2), col0

    def update(r0, c0, y):
        av = acc_val[pl.ds(r0, TILE_R), pl.ds(c0, TILE_L)]
        take = y > av
        ac = acc_chunk[pl.ds(r0, TILE_R), pl.ds(c0, TILE_L)]
        acc_chunk[pl.ds(r0, TILE_R), pl.ds(c0, TILE_L)] = jnp.where(take, j, ac)
        acc_val[pl.ds(r0, TILE_R), pl.ds(c0, TILE_L)] = jnp.maximum(av, y)

    @pl.when(j < NC - 1)
    def _main():
        def body(t, carry):
            r0, c0 = tile_at(t)
            y, _ = compute_y(r0, c0)
            update(r0, c0, y)
            return carry

        jax.lax.fori_loop(0, NT, body, 0)

    @pl.when(j == NC - 1)
    def _tail():
        def body(t, carry):
            r0, c0 = tile_at(t)
            y, col0 = compute_y(r0, c0)
            y = jnp.where(col0 + lane_i32 < V, y, NEG_INF)
            update(r0, c0, y)
            return carry

        jax.lax.fori_loop(0, NT, body, 0)

    @pl.when(j == NC - 1)
    def _finalize():
        vals = acc_val[...]
        row_max = jnp.max(vals, axis=1, keepdims=True)
        full_lane = jax.lax.broadcasted_iota(jnp.int32, (R, CHUNK), 1)
        cols = acc_chunk[...] * CHUNK + full_lane
        cand = jnp.where(vals == row_max, cols, BIG_I32)
        val_ref[...] = row_max
        idx_ref[...] = jnp.min(cand, axis=1, keepdims=True)


def kernel(logits, temperatures):
    t2 = temperatures.reshape(R, 1)
    _, idxs = pl.pallas_call(
        _sample_kernel,
        grid=(NC,),
        in_specs=[
            pl.BlockSpec((R, CHUNK), lambda j: (0, j)),
            pl.BlockSpec((R, 1), lambda j: (0, 0)),
        ],
        out_specs=[
            pl.BlockSpec((R, 1), lambda j: (0, 0)),
            pl.BlockSpec((R, 1), lambda j: (0, 0)),
        ],
        out_shape=[
            jax.ShapeDtypeStruct((R, 1), jnp.float32),
            jax.ShapeDtypeStruct((R, 1), jnp.int32),
        ],
        scratch_shapes=[
            pltpu.VMEM((R, CHUNK), jnp.float32),
            pltpu.VMEM((R, CHUNK), jnp.int32),
            pltpu.VMEM((R, 1), jnp.float32),
        ],
    )(logits, t2)
    return idxs.reshape(R)


# speedup vs baseline: 1.3312x; 1.0590x over previous
import jax
import jax.numpy as jnp
from jax.experimental import pallas as pl
from jax.experimental.pallas import tpu as pltpu

R = 128
V = 100000
CHUNK = 4096
NC = 25
TILE_R = 16
TILE_L = 512
NT = (R // TILE_R) * (CHUNK // TILE_L)

KEY1 = 42

NEG_INF = float("-inf")
BIG_I32 = 2**31 - 1
INV_LN2 = 1.4426950408889634


def _threefry_bits_key042(x1i):
    u32 = jnp.uint32
    ks0 = u32(0)
    ks1 = u32(KEY1)
    ks2 = u32(KEY1 ^ 0x1BD11BDA)
    ks = (ks0, ks1, ks2)
    rot1 = (13, 15, 26, 6)
    rot2 = (17, 29, 16, 24)

    x0 = x1i
    x1 = ((x1i << u32(13)) | (x1i >> u32(19))) ^ x0
    for r in rot1[1:]:
        x0 = x0 + x1
        x1 = (x1 << u32(r)) | (x1 >> u32(32 - r))
        x1 = x1 ^ x0
    x0 = x0 + ks[1]
    x1 = x1 + ks[2] + u32(1)
    for i in range(1, 5):
        for r in (rot1 if i % 2 == 0 else rot2):
            x0 = x0 + x1
            x1 = (x1 << u32(r)) | (x1 >> u32(32 - r))
            x1 = x1 ^ x0
        x0 = x0 + ks[(i + 1) % 3]
        x1 = x1 + ks[(i + 2) % 3] + u32(i + 1)
    return x0 ^ x1


def _threefry_bits_key042_x2(a1i, b1i):
    u32 = jnp.uint32
    ks = (u32(0), u32(KEY1), u32(KEY1 ^ 0x1BD11BDA))
    rot1 = (13, 15, 26, 6)
    rot2 = (17, 29, 16, 24)

    a0, b0 = a1i, b1i
    a1 = ((a1i << u32(13)) | (a1i >> u32(19))) ^ a0
    b1 = ((b1i << u32(13)) | (b1i >> u32(19))) ^ b0
    for r in rot1[1:]:
        a0 = a0 + a1
        b0 = b0 + b1
        a1 = (a1 << u32(r)) | (a1 >> u32(32 - r))
        b1 = (b1 << u32(r)) | (b1 >> u32(32 - r))
        a1 = a1 ^ a0
        b1 = b1 ^ b0
    a0 = a0 + ks[1]
    b0 = b0 + ks[1]
    a1 = a1 + ks[2] + u32(1)
    b1 = b1 + ks[2] + u32(1)
    for i in range(1, 5):
        for r in (rot1 if i % 2 == 0 else rot2):
            a0 = a0 + a1
            b0 = b0 + b1
            a1 = (a1 << u32(r)) | (a1 >> u32(32 - r))
            b1 = (b1 << u32(r)) | (b1 >> u32(32 - r))
            a1 = a1 ^ a0
            b1 = b1 ^ b0
        a0 = a0 + ks[(i + 1) % 3]
        b0 = b0 + ks[(i + 1) % 3]
        a1 = a1 + ks[(i + 2) % 3] + u32(i + 1)
        b1 = b1 + ks[(i + 2) % 3] + u32(i + 1)
    return a0 ^ a1, b0 ^ b1


def _sample_kernel(logits_ref, t_ref, val_ref, idx_ref,
                   acc_val, acc_chunk, s_ref):
    j = pl.program_id(0)

    @pl.when(j == 0)
    def _init():
        acc_val[...] = jnp.full((R, CHUNK), NEG_INF, jnp.float32)
        acc_chunk[...] = jnp.zeros((R, CHUNK), jnp.int32)
        s_ref[...] = jnp.float32(INV_LN2) / t_ref[...]

    row_iota = jax.lax.broadcasted_iota(jnp.uint32, (TILE_R, TILE_L), 0)
    lane_iota = jax.lax.broadcasted_iota(jnp.uint32, (TILE_R, TILE_L), 1)
    base = row_iota * jnp.uint32(V) + lane_iota
    lane_i32 = lane_iota.astype(jnp.int32)
    chunk_col0 = j * CHUNK

    def tile_at(t):
        r0 = (t >> 2) * TILE_R
        c0 = (t & 3) * (2 * TILE_L)
        return r0, c0

    def compute_y2(r0, c0):
        x1i_a = base + (r0 * V + chunk_col0 + c0 + KEY1).astype(jnp.uint32)
        x1i_b = x1i_a + jnp.uint32(TILE_L)
        bits_a, bits_b = _threefry_bits_key042_x2(x1i_a, x1i_b)
        s_tile = s_ref[pl.ds(r0, TILE_R), :]

        def finish(bits, c):
            xs = logits_ref[pl.ds(r0, TILE_R), pl.ds(c, TILE_L)]
            f = jax.lax.bitcast_convert_type(
                (bits >> jnp.uint32(9)) | jnp.uint32(0x3F800000), jnp.float32)
            n2 = jnp.maximum(-jnp.log2(2.0 - f), 1e-10)
            return xs * s_tile - jnp.log2(n2)

        return finish(bits_a, c0), finish(bits_b, c0 + TILE_L)

    def update(r0, c0, y):
        av = acc_val[pl.ds(r0, TILE_R), pl.ds(c0, TILE_L)]
        take = y > av
        ac = acc_chunk[pl.ds(r0, TILE_R), pl.ds(c0, TILE_L)]
        acc_chunk[pl.ds(r0, TILE_R), pl.ds(c0, TILE_L)] = jnp.where(take, j, ac)
        acc_val[pl.ds(r0, TILE_R), pl.ds(c0, TILE_L)] = jnp.maximum(av, y)

    @pl.when(j < NC - 1)
    def _main():
        def body(t, carry):
            r0, c0 = tile_at(t)
            ya, yb = compute_y2(r0, c0)
            update(r0, c0, ya)
            update(r0, c0 + TILE_L, yb)
            return carry

        jax.lax.fori_loop(0, NT // 2, body, 0)

    @pl.when(j == NC - 1)
    def _tail():
        def body(t, carry):
            r0, c0 = tile_at(t)
            ya, yb = compute_y2(r0, c0)
            ya = jnp.where(chunk_col0 + c0 + lane_i32 < V, ya, NEG_INF)
            yb = jnp.where(chunk_col0 + c0 + TILE_L + lane_i32 < V, yb, NEG_INF)
            update(r0, c0, ya)
            update(r0, c0 + TILE_L, yb)
            return carry

        jax.lax.fori_loop(0, NT // 2, body, 0)

    @pl.when(j == NC - 1)
    def _finalize():
        vals = acc_val[...]
        row_max = jnp.max(vals, axis=1, keepdims=True)
        full_lane = jax.lax.broadcasted_iota(jnp.int32, (R, CHUNK), 1)
        cols = acc_chunk[...] * CHUNK + full_lane
        cand = jnp.where(vals == row_max, cols, BIG_I32)
        val_ref[...] = row_max
        idx_ref[...] = jnp.min(cand, axis=1, keepdims=True)


def kernel(logits, temperatures):
    t2 = temperatures.reshape(R, 1)
    _, idxs = pl.pallas_call(
        _sample_kernel,
        grid=(NC,),
        in_specs=[
            pl.BlockSpec((R, CHUNK), lambda j: (0, j)),
            pl.BlockSpec((R, 1), lambda j: (0, 0)),
        ],
        out_specs=[
            pl.BlockSpec((R, 1), lambda j: (0, 0)),
            pl.BlockSpec((R, 1), lambda j: (0, 0)),
        ],
        out_shape=[
            jax.ShapeDtypeStruct((R, 1), jnp.float32),
            jax.ShapeDtypeStruct((R, 1), jnp.int32),
        ],
        scratch_shapes=[
            pltpu.VMEM((R, CHUNK), jnp.float32),
            pltpu.VMEM((R, CHUNK), jnp.int32),
            pltpu.VMEM((R, 1), jnp.float32),
        ],
    )(logits, t2)
    return idxs.reshape(R)
